# baseline (speedup 1.0000x reference)
import functools

import jax
import jax.numpy as jnp
from jax import lax
from jax.experimental import pallas as pl
from jax.experimental.pallas import tpu as pltpu
from jax.experimental.pallas import tpu_sc as plsc

_N = 10000
_E = 320000
_D = 128
_H = 128
_C = 64

_NC = 2
_NS = 16
_NW = _NC * _NS
_EPW = _E // _NW
_CH = 40
_NCHUNK = _EPW // _CH
_RPT = 624
_TAIL0 = _RPT * _NS
_TAILN = _N - _TAIL0
_DEGW = 128

def _deg_body(src_hbm, dst_hbm, ones_hbm, zeros_hbm, out_hbm,
              src_v, dst_v, src_v2, dst_v2, ones_v, acc,
              ix0, ix1, ss0, ss1):
    cid = lax.axis_index("c")
    sid = lax.axis_index("s")
    wid = sid * _NC + cid
    row0 = sid * _RPT
    pltpu.sync_copy(zeros_hbm.at[pl.ds(row0, _RPT)], acc.at[pl.ds(row0, _RPT)])

    @pl.when(sid == 0)
    def _():
        pltpu.sync_copy(zeros_hbm.at[pl.ds(_TAIL0, _TAILN)], acc.at[pl.ds(_TAIL0, _TAILN)])

    pltpu.sync_copy(ones_hbm, ones_v)
    plsc.subcore_barrier()

    base0 = wid * _EPW

    def idx_start(i, sv, dv, sem):
        base = base0 + i * _CH
        pltpu.async_copy(src_hbm.at[pl.ds(base, _CH)], sv, sem)
        pltpu.async_copy(dst_hbm.at[pl.ds(base, _CH)], dv, sem)

    def idx_wait(i, sv, dv, sem):
        base = base0 + i * _CH
        pltpu.make_async_copy(src_hbm.at[pl.ds(base, _CH)], sv, sem).wait()
        pltpu.make_async_copy(dst_hbm.at[pl.ds(base, _CH)], dv, sem).wait()

    def pair_wait(sv, dv, sem):
        pltpu.make_async_copy(ones_v.at[0], acc.at[sv], sem).wait()
        pltpu.make_async_copy(ones_v.at[1], acc.at[dv], sem).wait()

    idx_start(0, src_v, dst_v, ix0)
    idx_start(1, src_v2, dst_v2, ix1)

    def half(i, sv, dv, ixs, ss, nsv, ndv, nixs, ns):
        idx_wait(i, sv, dv, ixs)
        pltpu.async_copy(ones_v.at[0], acc.at[sv], ss, add=True)
        pltpu.async_copy(ones_v.at[1], acc.at[dv], ss, add=True)

        @pl.when(i >= 1)
        def _():
            pair_wait(nsv, ndv, ns)

            @pl.when(i + 1 < _NCHUNK)
            def _():
                idx_start(i + 1, nsv, ndv, nixs)

    def body(k, carry):
        i0 = 2 * k
        half(i0, src_v, dst_v, ix0, ss0, src_v2, dst_v2, ix1, ss1)
        half(i0 + 1, src_v2, dst_v2, ix1, ss1, src_v, dst_v, ix0, ss0)
        return carry

    lax.fori_loop(0, _NCHUNK // 2, body, 0)
    pair_wait(src_v2, dst_v2, ss1)
    plsc.subcore_barrier()
    pltpu.sync_copy(acc.at[pl.ds(row0, _RPT)], out_hbm.at[cid, pl.ds(row0, _RPT)])

    @pl.when(sid == 0)
    def _():
        pltpu.sync_copy(acc.at[pl.ds(_TAIL0, _TAILN)], out_hbm.at[cid, pl.ds(_TAIL0, _TAILN)])


@functools.cache
def _deg_call():
    mesh = plsc.VectorSubcoreMesh(
        core_axis_name="c", subcore_axis_name="s",
        num_cores=_NC, num_subcores=_NS,
    )
    return pl.kernel(
        _deg_body,
        out_type=jax.ShapeDtypeStruct((_NC, _N, _DEGW), jnp.float32),
        mesh=mesh,
        scratch_types=[
            pltpu.VMEM((_CH,), jnp.int32),
            pltpu.VMEM((_CH,), jnp.int32),
            pltpu.VMEM((_CH,), jnp.int32),
            pltpu.VMEM((_CH,), jnp.int32),
            pltpu.VMEM((2, _CH, _DEGW), jnp.float32),
            pltpu.VMEM_SHARED((_N, _DEGW), jnp.float32),
            pltpu.SemaphoreType.DMA,
            pltpu.SemaphoreType.DMA,
            pltpu.SemaphoreType.DMA,
            pltpu.SemaphoreType.DMA,
        ],
    )


_MCH = 128
_MPH = 2
_MNCH = 40
_NA = _N + 8


def _mp_body(h_hbm, src_hbm, dst_hbm, zeros_hbm, out_hbm,
             srcw, dstw, rows0, rows1, acc, g0, g1, s0, s1):
    cid = lax.axis_index("c")
    sid = lax.axis_index("s")
    wid = sid * _NC + cid
    row0 = sid * _RPT
    pltpu.sync_copy(zeros_hbm.at[pl.ds(row0, _RPT)], acc.at[pl.ds(row0, _RPT)])

    @pl.when(sid == 0)
    def _():
        pltpu.sync_copy(zeros_hbm.at[pl.ds(_TAIL0, _TAILN)], acc.at[pl.ds(_TAIL0, _TAILN)])

    def body(k, carry):
        i0 = 2 * k
        pltpu.make_async_copy(h_hbm.at[srcw.at[i0]], rows0, g0).wait()
        pltpu.async_copy(rows0, acc.at[dstw.at[i0]], s0, add=True)
        pltpu.make_async_copy(h_hbm.at[srcw.at[i0 + 1]], rows1, g1).wait()
        pltpu.async_copy(rows1, acc.at[dstw.at[i0 + 1]], s1, add=True)

        @pl.when(i0 + 2 < _MNCH)
        def _():
            pltpu.make_async_copy(rows0, acc.at[dstw.at[i0]], s0).wait()
            pltpu.async_copy(h_hbm.at[srcw.at[i0 + 2]], rows0, g0)

        @pl.when(i0 + 3 < _MNCH)
        def _():
            pltpu.make_async_copy(rows1, acc.at[dstw.at[i0 + 1]], s1).wait()
            pltpu.async_copy(h_hbm.at[srcw.at[i0 + 3]], rows1, g1)

        return carry

    for p in range(_MPH):
        pltpu.sync_copy(src_hbm.at[wid, p], srcw)
        pltpu.sync_copy(dst_hbm.at[wid, p], dstw)
        if p == 0:
            plsc.subcore_barrier()
        pltpu.async_copy(h_hbm.at[srcw.at[0]], rows0, g0)
        pltpu.async_copy(h_hbm.at[srcw.at[1]], rows1, g1)
        lax.fori_loop(0, _MNCH // 2, body, 0)
        pltpu.make_async_copy(rows0, acc.at[dstw.at[0]], s0).wait()
        pltpu.make_async_copy(rows1, acc.at[dstw.at[1]], s1).wait()

    plsc.subcore_barrier()
    pltpu.sync_copy(acc.at[pl.ds(row0, _RPT)], out_hbm.at[cid, pl.ds(row0, _RPT)])

    @pl.when(sid == 0)
    def _():
        pltpu.sync_copy(acc.at[pl.ds(_TAIL0, _TAILN)], out_hbm.at[cid, pl.ds(_TAIL0, _TAILN)])


@functools.cache
def _mp_call():
    mesh = plsc.VectorSubcoreMesh(
        core_axis_name="c", subcore_axis_name="s",
        num_cores=_NC, num_subcores=_NS,
    )
    return pl.kernel(
        _mp_body,
        out_type=jax.ShapeDtypeStruct((_NC, _N, _H), jnp.float32),
        mesh=mesh,
        scratch_types=[
            pltpu.VMEM((_MNCH, _MCH), jnp.int32),
            pltpu.VMEM((_MNCH, _MCH), jnp.int32),
            pltpu.VMEM((_MCH, _H), jnp.float32),
            pltpu.VMEM((_MCH, _H), jnp.float32),
            pltpu.VMEM_SHARED((_NA, _H), jnp.float32),
            pltpu.SemaphoreType.DMA,
            pltpu.SemaphoreType.DMA,
            pltpu.SemaphoreType.DMA,
            pltpu.SemaphoreType.DMA,
        ],
    )


_RB = 1000
_NGRID = _N // _RB


def _dense1_body(x_ref, w_ref, dego_ref, out_ref):
    ns = lax.rsqrt(jnp.maximum(dego_ref[0] + dego_ref[1], 1.0))
    out_ref[...] = jnp.dot(x_ref[...], w_ref[...],
                           preferred_element_type=jnp.float32) * ns


def _dense1(x, W, dego_p):
    return pl.pallas_call(
        _dense1_body,
        grid=(_NGRID,),
        in_specs=[
            pl.BlockSpec((_RB, _D), lambda j: (j, 0)),
            pl.BlockSpec((_D, _H), lambda j: (0, 0)),
            pl.BlockSpec((2, _RB, 1), lambda j: (0, j, 0)),
        ],
        out_specs=pl.BlockSpec((_RB, _H), lambda j: (j, 0)),
        out_shape=jax.ShapeDtypeStruct((_N, _H), jnp.float32),
    )(x, W, dego_p)


def _dense2_body(p_ref, degi_ref, dego_ref, b1_ref, w_ref, out_ref):
    agg = p_ref[0] + p_ref[1]
    nd = lax.rsqrt(jnp.maximum(degi_ref[0] + degi_ref[1], 1.0))
    h = agg * nd + b1_ref[...]
    h = jnp.where(h > 0, h, jnp.exp(h) - 1.0)
    ns = lax.rsqrt(jnp.maximum(dego_ref[0] + dego_ref[1], 1.0))
    out_ref[...] = jnp.dot(h, w_ref[...],
                           preferred_element_type=jnp.float32) * ns


def _dense2(p, degi_p, dego_p, b1, W):
    return pl.pallas_call(
        _dense2_body,
        grid=(_NGRID,),
        in_specs=[
            pl.BlockSpec((2, _RB, _H), lambda j: (0, j, 0)),
            pl.BlockSpec((2, _RB, 1), lambda j: (0, j, 0)),
            pl.BlockSpec((2, _RB, 1), lambda j: (0, j, 0)),
            pl.BlockSpec((1, _H), lambda j: (0, 0)),
            pl.BlockSpec((_H, _H), lambda j: (0, 0)),
        ],
        out_specs=pl.BlockSpec((_RB, _H), lambda j: (j, 0)),
        out_shape=jax.ShapeDtypeStruct((_N, _H), jnp.float32),
    )(p, degi_p, dego_p, b1, W)


def _final_body(p_ref, degi_ref, b2_ref, wc_ref, bc_ref, out_ref, acc_ref):
    j = pl.program_id(0)

    @pl.when(j == 0)
    def _():
        acc_ref[...] = jnp.zeros_like(acc_ref)

    agg = p_ref[0] + p_ref[1]
    nd = lax.rsqrt(jnp.maximum(degi_ref[0] + degi_ref[1], 1.0))
    z = agg * nd + b2_ref[...]
    z = jnp.where(z > 0, z, jnp.exp(z) - 1.0)
    acc_ref[...] += jnp.sum(z, axis=0, keepdims=True)

    @pl.when(j == pl.num_programs(0) - 1)
    def _():
        out_ref[...] = jnp.dot(acc_ref[...] * (1.0 / _N), wc_ref[...],
                               preferred_element_type=jnp.float32) + bc_ref[...]


def _final(p, degi_p, b2, Wc, bc):
    return pl.pallas_call(
        _final_body,
        grid=(_NGRID,),
        in_specs=[
            pl.BlockSpec((2, _RB, _H), lambda j: (0, j, 0)),
            pl.BlockSpec((2, _RB, 1), lambda j: (0, j, 0)),
            pl.BlockSpec((1, _H), lambda j: (0, 0)),
            pl.BlockSpec((_H, _C), lambda j: (0, 0)),
            pl.BlockSpec((1, _C), lambda j: (0, 0)),
        ],
        out_specs=pl.BlockSpec((1, _C), lambda j: (0, 0)),
        out_shape=jax.ShapeDtypeStruct((1, _C), jnp.float32),
        scratch_shapes=[pltpu.VMEM((1, _H), jnp.float32)],
    )(p, degi_p, b2, Wc, bc)


def kernel(features, edge_index, W1, b1, W2, b2, Wc, bc):
    src = edge_index[0]
    dst = edge_index[1]

    lane = jnp.arange(_DEGW, dtype=jnp.int32)
    ones = jnp.stack([
        jnp.where(lane == 0, 1.0, 0.0),
        jnp.where(lane == 1, 1.0, 0.0),
    ]).astype(jnp.float32)[:, None, :] * jnp.ones((2, _CH, _DEGW), jnp.float32)
    zeros_h = jnp.zeros((_N, _H), jnp.float32)
    zeros_d = jnp.zeros((_N, _DEGW), jnp.float32)

    degp = _deg_call()(src, dst, ones, zeros_d)
    dego_p = degp[:, :, 0:1]
    degi_p = degp[:, :, 1:2]

    b1r = b1.reshape(1, _H)
    b2r = b2.reshape(1, _H)
    bcr = bc.reshape(1, _C)

    npad = _MPH * _MNCH * _MCH - _EPW
    src3 = jnp.concatenate(
        [src.reshape(_NW, _EPW), jnp.zeros((_NW, npad), jnp.int32)], 1
    ).reshape(_NW, _MPH, _MNCH, _MCH)
    dst3 = jnp.concatenate(
        [dst.reshape(_NW, _EPW), jnp.full((_NW, npad), _N, jnp.int32)], 1
    ).reshape(_NW, _MPH, _MNCH, _MCH)

    hs1 = _dense1(features, W1, dego_p)
    p1 = _mp_call()(hs1, src3, dst3, zeros_h)
    hs2 = _dense2(p1, degi_p, dego_p, b1r, W2)
    p2 = _mp_call()(hs2, src3, dst3, zeros_h)
    return _final(p2, degi_p, b2r, Wc, bcr)


# speedup vs baseline: 2.1172x; 2.1172x over previous
import functools

import jax
import jax.numpy as jnp
from jax import lax
from jax.experimental import pallas as pl
from jax.experimental.pallas import tpu as pltpu
from jax.experimental.pallas import tpu_sc as plsc

_N = 10000
_E = 320000
_D = 128
_H = 128
_C = 64

_NC = 2
_NS = 16
_NW = _NC * _NS
_EPW = _E // _NW
_CH = 40
_NCHUNK = _EPW // _CH
_RPT = 624
_TAIL0 = _RPT * _NS
_TAILN = _N - _TAIL0
_DEGW = 128

def _deg_body(src_hbm, dst_hbm, ones_hbm, zeros_hbm, out_hbm,
              src_v, dst_v, src_v2, dst_v2, ones_v, acc,
              ix0, ix1, ss0, ss1):
    cid = lax.axis_index("c")
    sid = lax.axis_index("s")
    wid = sid * _NC + cid
    row0 = sid * _RPT
    pltpu.sync_copy(zeros_hbm.at[pl.ds(row0, _RPT)], acc.at[pl.ds(row0, _RPT)])

    @pl.when(sid == 0)
    def _():
        pltpu.sync_copy(zeros_hbm.at[pl.ds(_TAIL0, _TAILN)], acc.at[pl.ds(_TAIL0, _TAILN)])

    pltpu.sync_copy(ones_hbm, ones_v)
    plsc.subcore_barrier()

    base0 = wid * _EPW

    def idx_start(i, sv, dv, sem):
        base = base0 + i * _CH
        pltpu.async_copy(src_hbm.at[pl.ds(base, _CH)], sv, sem)
        pltpu.async_copy(dst_hbm.at[pl.ds(base, _CH)], dv, sem)

    def idx_wait(i, sv, dv, sem):
        base = base0 + i * _CH
        pltpu.make_async_copy(src_hbm.at[pl.ds(base, _CH)], sv, sem).wait()
        pltpu.make_async_copy(dst_hbm.at[pl.ds(base, _CH)], dv, sem).wait()

    def pair_wait(sv, dv, sem):
        pltpu.make_async_copy(ones_v.at[0], acc.at[sv], sem).wait()
        pltpu.make_async_copy(ones_v.at[1], acc.at[dv], sem).wait()

    idx_start(0, src_v, dst_v, ix0)
    idx_start(1, src_v2, dst_v2, ix1)

    def half(i, sv, dv, ixs, ss, nsv, ndv, nixs, ns):
        idx_wait(i, sv, dv, ixs)
        pltpu.async_copy(ones_v.at[0], acc.at[sv], ss, add=True)
        pltpu.async_copy(ones_v.at[1], acc.at[dv], ss, add=True)

        @pl.when(i >= 1)
        def _():
            pair_wait(nsv, ndv, ns)

            @pl.when(i + 1 < _NCHUNK)
            def _():
                idx_start(i + 1, nsv, ndv, nixs)

    def body(k, carry):
        i0 = 2 * k
        half(i0, src_v, dst_v, ix0, ss0, src_v2, dst_v2, ix1, ss1)
        half(i0 + 1, src_v2, dst_v2, ix1, ss1, src_v, dst_v, ix0, ss0)
        return carry

    lax.fori_loop(0, _NCHUNK // 2, body, 0)
    pair_wait(src_v2, dst_v2, ss1)
    plsc.subcore_barrier()
    pltpu.sync_copy(acc.at[pl.ds(row0, _RPT)], out_hbm.at[cid, pl.ds(row0, _RPT)])

    @pl.when(sid == 0)
    def _():
        pltpu.sync_copy(acc.at[pl.ds(_TAIL0, _TAILN)], out_hbm.at[cid, pl.ds(_TAIL0, _TAILN)])


@functools.cache
def _deg_call():
    mesh = plsc.VectorSubcoreMesh(
        core_axis_name="c", subcore_axis_name="s",
        num_cores=_NC, num_subcores=_NS,
    )
    return pl.kernel(
        _deg_body,
        out_type=jax.ShapeDtypeStruct((_NC, _N, _DEGW), jnp.float32),
        mesh=mesh,
        scratch_types=[
            pltpu.VMEM((_CH,), jnp.int32),
            pltpu.VMEM((_CH,), jnp.int32),
            pltpu.VMEM((_CH,), jnp.int32),
            pltpu.VMEM((_CH,), jnp.int32),
            pltpu.VMEM((2, _CH, _DEGW), jnp.float32),
            pltpu.VMEM_SHARED((_N, _DEGW), jnp.float32),
            pltpu.SemaphoreType.DMA,
            pltpu.SemaphoreType.DMA,
            pltpu.SemaphoreType.DMA,
            pltpu.SemaphoreType.DMA,
        ],
    )


_MCH = 40
_MNCH = 250


def _mp_body(h_hbm, src_hbm, dst_hbm, zeros_hbm, out_hbm,
             sv0, sv1, sv2, sv3,
             dv0a, dv1a, dv2a, dv3a, dv0b, dv1b, dv2b, dv3b,
             rows0, rows1, rows2, rows3, acc,
             i0s, i1s, i2s, i3s, g0, g1, g2, g3, s0, s1, s2, s3):
    cid = lax.axis_index("c")
    sid = lax.axis_index("s")
    wid = sid * _NC + cid
    row0 = sid * _RPT
    pltpu.sync_copy(zeros_hbm.at[pl.ds(row0, _RPT)], acc.at[pl.ds(row0, _RPT)])

    @pl.when(sid == 0)
    def _():
        pltpu.sync_copy(zeros_hbm.at[pl.ds(_TAIL0, _TAILN)], acc.at[pl.ds(_TAIL0, _TAILN)])

    plsc.subcore_barrier()

    S0 = (sv0, (dv0a, dv0b), rows0, i0s, g0, s0)
    S1 = (sv1, (dv1a, dv1b), rows1, i1s, g1, s1)
    S2 = (sv2, (dv2a, dv2b), rows2, i2s, g2, s2)
    S3 = (sv3, (dv3a, dv3b), rows3, i3s, g3, s3)

    def iissue(i, slot, gen):
        pltpu.async_copy(src_hbm.at[wid, i], slot[0], slot[3])
        pltpu.async_copy(dst_hbm.at[wid, i], slot[1][gen], slot[3])

    def iwait(i, slot, gen):
        pltpu.make_async_copy(src_hbm.at[wid, i], slot[0], slot[3]).wait()
        pltpu.make_async_copy(dst_hbm.at[wid, i], slot[1][gen], slot[3]).wait()

    def gissue(slot):
        pltpu.async_copy(h_hbm.at[slot[0]], slot[2], slot[4])

    def gwait(slot):
        pltpu.make_async_copy(h_hbm.at[slot[0]], slot[2], slot[4]).wait()

    def sissue(slot, gen):
        pltpu.async_copy(slot[2], acc.at[slot[1][gen]], slot[5], add=True)

    def swait(slot, gen):
        pltpu.make_async_copy(slot[2], acc.at[slot[1][gen]], slot[5]).wait()

    iissue(0, S0, 0)
    iissue(1, S1, 0)
    iissue(2, S2, 0)
    iissue(3, S3, 0)
    iwait(0, S0, 0)
    gissue(S0)
    iwait(1, S1, 0)
    gissue(S1)

    def pair(k, pa, pb, qa, qb, gen_p, gen_qprev, gen_iw):
        i0 = 2 * k
        gwait(pa)
        sissue(pa, gen_p)
        gwait(pb)
        sissue(pb, gen_p)

        @pl.when(i0 + 2 < _MNCH)
        def _():
            @pl.when(k >= 1)
            def _():
                swait(qa, gen_qprev)
                swait(qb, gen_qprev)

            iwait(i0 + 2, qa, gen_iw)
            iwait(i0 + 3, qb, gen_iw)
            gissue(qa)
            gissue(qb)

        @pl.when(i0 + 4 < _MNCH)
        def _():
            iissue(i0 + 4, pa, 1 - gen_p)
            iissue(i0 + 5, pb, 1 - gen_p)

    def body(k, carry):
        m = k % 4

        @pl.when(m == 0)
        def _():
            pair(k, S0, S1, S2, S3, 0, 1, 0)

        @pl.when(m == 1)
        def _():
            pair(k, S2, S3, S0, S1, 0, 0, 1)

        @pl.when(m == 2)
        def _():
            pair(k, S0, S1, S2, S3, 1, 0, 1)

        @pl.when(m == 3)
        def _():
            pair(k, S2, S3, S0, S1, 1, 1, 0)

        return carry

    lax.fori_loop(0, _MNCH // 2, body, 0)
    swait(S2, 1)
    swait(S3, 1)
    swait(S0, 0)
    swait(S1, 0)
    plsc.subcore_barrier()
    pltpu.sync_copy(acc.at[pl.ds(row0, _RPT)], out_hbm.at[cid, pl.ds(row0, _RPT)])

    @pl.when(sid == 0)
    def _():
        pltpu.sync_copy(acc.at[pl.ds(_TAIL0, _TAILN)], out_hbm.at[cid, pl.ds(_TAIL0, _TAILN)])


@functools.cache
def _mp_call():
    mesh = plsc.VectorSubcoreMesh(
        core_axis_name="c", subcore_axis_name="s",
        num_cores=_NC, num_subcores=_NS,
    )
    idx = pltpu.VMEM((_MCH,), jnp.int32)
    rowb = pltpu.VMEM((_MCH, _H), jnp.float32)
    sem = pltpu.SemaphoreType.DMA
    return pl.kernel(
        _mp_body,
        out_type=jax.ShapeDtypeStruct((_NC, _N, _H), jnp.float32),
        mesh=mesh,
        scratch_types=(
            [idx] * 12 + [rowb] * 4
            + [pltpu.VMEM_SHARED((_N, _H), jnp.float32)]
            + [sem] * 12
        ),
    )


_RB = 1000
_NGRID = _N // _RB


def _dense1_body(x_ref, w_ref, dego_ref, out_ref):
    ns = lax.rsqrt(jnp.maximum(dego_ref[0] + dego_ref[1], 1.0))
    out_ref[...] = jnp.dot(x_ref[...], w_ref[...],
                           preferred_element_type=jnp.float32) * ns


def _dense1(x, W, dego_p):
    return pl.pallas_call(
        _dense1_body,
        grid=(_NGRID,),
        in_specs=[
            pl.BlockSpec((_RB, _D), lambda j: (j, 0)),
            pl.BlockSpec((_D, _H), lambda j: (0, 0)),
            pl.BlockSpec((2, _RB, 1), lambda j: (0, j, 0)),
        ],
        out_specs=pl.BlockSpec((_RB, _H), lambda j: (j, 0)),
        out_shape=jax.ShapeDtypeStruct((_N, _H), jnp.float32),
    )(x, W, dego_p)


def _dense2_body(p_ref, degi_ref, dego_ref, b1_ref, w_ref, out_ref):
    agg = p_ref[0] + p_ref[1]
    nd = lax.rsqrt(jnp.maximum(degi_ref[0] + degi_ref[1], 1.0))
    h = agg * nd + b1_ref[...]
    h = jnp.where(h > 0, h, jnp.exp(h) - 1.0)
    ns = lax.rsqrt(jnp.maximum(dego_ref[0] + dego_ref[1], 1.0))
    out_ref[...] = jnp.dot(h, w_ref[...],
                           preferred_element_type=jnp.float32) * ns


def _dense2(p, degi_p, dego_p, b1, W):
    return pl.pallas_call(
        _dense2_body,
        grid=(_NGRID,),
        in_specs=[
            pl.BlockSpec((2, _RB, _H), lambda j: (0, j, 0)),
            pl.BlockSpec((2, _RB, 1), lambda j: (0, j, 0)),
            pl.BlockSpec((2, _RB, 1), lambda j: (0, j, 0)),
            pl.BlockSpec((1, _H), lambda j: (0, 0)),
            pl.BlockSpec((_H, _H), lambda j: (0, 0)),
        ],
        out_specs=pl.BlockSpec((_RB, _H), lambda j: (j, 0)),
        out_shape=jax.ShapeDtypeStruct((_N, _H), jnp.float32),
    )(p, degi_p, dego_p, b1, W)


def _final_body(p_ref, degi_ref, b2_ref, wc_ref, bc_ref, out_ref, acc_ref):
    j = pl.program_id(0)

    @pl.when(j == 0)
    def _():
        acc_ref[...] = jnp.zeros_like(acc_ref)

    agg = p_ref[0] + p_ref[1]
    nd = lax.rsqrt(jnp.maximum(degi_ref[0] + degi_ref[1], 1.0))
    z = agg * nd + b2_ref[...]
    z = jnp.where(z > 0, z, jnp.exp(z) - 1.0)
    acc_ref[...] += jnp.sum(z, axis=0, keepdims=True)

    @pl.when(j == pl.num_programs(0) - 1)
    def _():
        out_ref[...] = jnp.dot(acc_ref[...] * (1.0 / _N), wc_ref[...],
                               preferred_element_type=jnp.float32) + bc_ref[...]


def _final(p, degi_p, b2, Wc, bc):
    return pl.pallas_call(
        _final_body,
        grid=(_NGRID,),
        in_specs=[
            pl.BlockSpec((2, _RB, _H), lambda j: (0, j, 0)),
            pl.BlockSpec((2, _RB, 1), lambda j: (0, j, 0)),
            pl.BlockSpec((1, _H), lambda j: (0, 0)),
            pl.BlockSpec((_H, _C), lambda j: (0, 0)),
            pl.BlockSpec((1, _C), lambda j: (0, 0)),
        ],
        out_specs=pl.BlockSpec((1, _C), lambda j: (0, 0)),
        out_shape=jax.ShapeDtypeStruct((1, _C), jnp.float32),
        scratch_shapes=[pltpu.VMEM((1, _H), jnp.float32)],
    )(p, degi_p, b2, Wc, bc)


def kernel(features, edge_index, W1, b1, W2, b2, Wc, bc):
    src = edge_index[0]
    dst = edge_index[1]

    lane = jnp.arange(_DEGW, dtype=jnp.int32)
    ones = jnp.stack([
        jnp.where(lane == 0, 1.0, 0.0),
        jnp.where(lane == 1, 1.0, 0.0),
    ]).astype(jnp.float32)[:, None, :] * jnp.ones((2, _CH, _DEGW), jnp.float32)
    zeros_h = jnp.zeros((_N, _H), jnp.float32)
    zeros_d = jnp.zeros((_N, _DEGW), jnp.float32)

    degp = _deg_call()(src, dst, ones, zeros_d)
    dego_p = degp[:, :, 0:1]
    degi_p = degp[:, :, 1:2]

    b1r = b1.reshape(1, _H)
    b2r = b2.reshape(1, _H)
    bcr = bc.reshape(1, _C)

    src3 = src.reshape(_NW, _MNCH, _MCH)
    dst3 = dst.reshape(_NW, _MNCH, _MCH)

    hs1 = _dense1(features, W1, dego_p)
    p1 = _mp_call()(hs1, src3, dst3, zeros_h)
    hs2 = _dense2(p1, degi_p, dego_p, b1r, W2)
    p2 = _mp_call()(hs2, src3, dst3, zeros_h)
    return _final(p2, degi_p, b2r, Wc, bcr)
